# baseline (speedup 1.0000x reference)
import jax
import jax.numpy as jnp
from jax.experimental import pallas as pl
from jax.experimental.pallas import tpu as pltpu

_BM = 400
_TBM = 80
_NMAIN = 23
_SLOTS = 5
_PER_STEP = _BM // _TBM


def _gcn_kernel(x_ref, w_ref, adj_ref, adj_any, out_ref, support_ref,
                tailbuf, tsem):
    i = pl.program_id(0)
    base = _BM * _NMAIN
    n_tail = 2 * _PER_STEP

    def tail_copy(j):
        slot = j % _SLOTS
        return pltpu.make_async_copy(
            adj_any.at[pl.ds(base + _TBM * j, _TBM), :],
            tailbuf.at[slot], tsem.at[slot])

    def tail_dots(j0):
        for j in range(j0, j0 + _PER_STEP):
            slot = j % _SLOTS
            tail_copy(j).wait()
            res = jax.lax.dot(
                tailbuf[slot].astype(jnp.bfloat16),
                support_ref[...],
                preferred_element_type=jnp.float32,
            )
            if j + _SLOTS < n_tail:
                tail_copy(j + _SLOTS).start()
            out_ref[pl.ds(_TBM * (j - j0), _TBM), :] = res

    @pl.when(i == _NMAIN - 1)
    def _():
        for j in range(_SLOTS):
            tail_copy(j).start()

    @pl.when(i == 0)
    def _():
        support_ref[...] = jax.lax.dot(
            x_ref[...].astype(jnp.bfloat16),
            w_ref[...].astype(jnp.bfloat16),
            preferred_element_type=jnp.float32,
        ).astype(jnp.bfloat16)

    @pl.when(i < _NMAIN)
    def _():
        out_ref[...] = jax.lax.dot(
            adj_ref[...].astype(jnp.bfloat16),
            support_ref[...],
            preferred_element_type=jnp.float32,
        )

    @pl.when(i == _NMAIN)
    def _():
        tail_dots(0)

    @pl.when(i == _NMAIN + 1)
    def _():
        tail_dots(_PER_STEP)


def kernel(input, adj, W):
    n, d_in = input.shape
    d_out = W.shape[1]
    grid = (_NMAIN + 2,)
    return pl.pallas_call(
        _gcn_kernel,
        grid=grid,
        in_specs=[
            pl.BlockSpec((n, d_in), lambda i: (0, 0)),
            pl.BlockSpec((d_in, d_out), lambda i: (0, 0)),
            pl.BlockSpec((_BM, n), lambda i: (jnp.minimum(i, _NMAIN - 1), 0)),
            pl.BlockSpec(memory_space=pltpu.MemorySpace.HBM),
        ],
        out_specs=pl.BlockSpec((_BM, d_out), lambda i: (i, 0)),
        out_shape=jax.ShapeDtypeStruct((n, d_out), jnp.float32),
        scratch_shapes=[
            pltpu.VMEM((n, d_out), jnp.bfloat16),
            pltpu.VMEM((_SLOTS, _TBM, n), jnp.float32),
            pltpu.SemaphoreType.DMA((_SLOTS,)),
        ],
        compiler_params=pltpu.CompilerParams(vmem_limit_bytes=67108864),
    )(input, W, adj, adj)


# speedup vs baseline: 1.0091x; 1.0091x over previous
import jax
import jax.numpy as jnp
from jax.experimental import pallas as pl
from jax.experimental.pallas import tpu as pltpu

_BM = 400
_TBM = 80
_NMAIN = 24
_SLOTS = 4


def _gcn_kernel(x_ref, w_ref, adj_ref, adj_any, out_ref, support_ref,
                tailbuf, tsem):
    i = pl.program_id(0)
    base = _BM * _NMAIN

    def tail_copy(j):
        slot = j % _SLOTS
        return pltpu.make_async_copy(
            adj_any.at[pl.ds(base + _TBM * j, _TBM), :],
            tailbuf.at[slot], tsem.at[slot])

    @pl.when(i == _NMAIN - 1)
    def _():
        for j in range(_SLOTS):
            tail_copy(j).start()

    @pl.when(i == 0)
    def _():
        support_ref[...] = jax.lax.dot(
            x_ref[...].astype(jnp.bfloat16),
            w_ref[...].astype(jnp.bfloat16),
            preferred_element_type=jnp.float32,
        ).astype(jnp.bfloat16)

    @pl.when(i < _NMAIN)
    def _():
        out_ref[...] = jax.lax.dot(
            adj_ref[...].astype(jnp.bfloat16),
            support_ref[...],
            preferred_element_type=jnp.float32,
        )

    @pl.when(i == _NMAIN)
    def _():
        n_tail = _BM // _TBM
        for j in range(n_tail):
            slot = j % _SLOTS
            tail_copy(j).wait()
            res = jax.lax.dot(
                tailbuf[slot].astype(jnp.bfloat16),
                support_ref[...],
                preferred_element_type=jnp.float32,
            )
            if j + _SLOTS < n_tail:
                tail_copy(j + _SLOTS).start()
            out_ref[pl.ds(_TBM * j, _TBM), :] = res


def kernel(input, adj, W):
    n, d_in = input.shape
    d_out = W.shape[1]
    grid = (_NMAIN + 1,)
    return pl.pallas_call(
        _gcn_kernel,
        grid=grid,
        in_specs=[
            pl.BlockSpec((n, d_in), lambda i: (0, 0)),
            pl.BlockSpec((d_in, d_out), lambda i: (0, 0)),
            pl.BlockSpec((_BM, n), lambda i: (jnp.minimum(i, _NMAIN - 1), 0)),
            pl.BlockSpec(memory_space=pltpu.MemorySpace.HBM),
        ],
        out_specs=pl.BlockSpec((_BM, d_out), lambda i: (i, 0)),
        out_shape=jax.ShapeDtypeStruct((n, d_out), jnp.float32),
        scratch_shapes=[
            pltpu.VMEM((n, d_out), jnp.bfloat16),
            pltpu.VMEM((_SLOTS, _TBM, n), jnp.float32),
            pltpu.SemaphoreType.DMA((_SLOTS,)),
        ],
        compiler_params=pltpu.CompilerParams(vmem_limit_bytes=67108864),
    )(input, W, adj, adj)


# speedup vs baseline: 1.0105x; 1.0014x over previous
import jax
import jax.numpy as jnp
from jax.experimental import pallas as pl
from jax.experimental.pallas import tpu as pltpu

_BM = 400
_TBM = 80
_NMAIN = 24
_SLOTS = 5


def _gcn_kernel(x_ref, w_ref, adj_ref, adj_any, out_ref, support_ref,
                tailbuf, tsem):
    i = pl.program_id(0)
    base = _BM * _NMAIN

    def tail_copy(j):
        slot = j % _SLOTS
        return pltpu.make_async_copy(
            adj_any.at[pl.ds(base + _TBM * j, _TBM), :],
            tailbuf.at[slot], tsem.at[slot])

    @pl.when(i == _NMAIN - 1)
    def _():
        for j in range(_SLOTS):
            tail_copy(j).start()

    @pl.when(i == 0)
    def _():
        support_ref[...] = jax.lax.dot(
            x_ref[...].astype(jnp.bfloat16),
            w_ref[...].astype(jnp.bfloat16),
            preferred_element_type=jnp.float32,
        ).astype(jnp.bfloat16)

    @pl.when(i < _NMAIN)
    def _():
        out_ref[...] = jax.lax.dot(
            adj_ref[...].astype(jnp.bfloat16),
            support_ref[...],
            preferred_element_type=jnp.float32,
        )

    @pl.when(i == _NMAIN)
    def _():
        n_tail = _BM // _TBM
        for j in range(n_tail):
            slot = j % _SLOTS
            tail_copy(j).wait()
            res = jax.lax.dot(
                tailbuf[slot].astype(jnp.bfloat16),
                support_ref[...],
                preferred_element_type=jnp.float32,
            )
            if j + _SLOTS < n_tail:
                tail_copy(j + _SLOTS).start()
            out_ref[pl.ds(_TBM * j, _TBM), :] = res


def kernel(input, adj, W):
    n, d_in = input.shape
    d_out = W.shape[1]
    grid = (_NMAIN + 1,)
    return pl.pallas_call(
        _gcn_kernel,
        grid=grid,
        in_specs=[
            pl.BlockSpec((n, d_in), lambda i: (0, 0)),
            pl.BlockSpec((d_in, d_out), lambda i: (0, 0)),
            pl.BlockSpec((_BM, n), lambda i: (jnp.minimum(i, _NMAIN - 1), 0)),
            pl.BlockSpec(memory_space=pltpu.MemorySpace.HBM),
        ],
        out_specs=pl.BlockSpec((_BM, d_out), lambda i: (i, 0)),
        out_shape=jax.ShapeDtypeStruct((n, d_out), jnp.float32),
        scratch_shapes=[
            pltpu.VMEM((n, d_out), jnp.bfloat16),
            pltpu.VMEM((_SLOTS, _TBM, n), jnp.float32),
            pltpu.SemaphoreType.DMA((_SLOTS,)),
        ],
        compiler_params=pltpu.CompilerParams(vmem_limit_bytes=67108864),
    )(input, W, adj, adj)
